# baseline (speedup 1.0000x reference)
import functools

import jax
import jax.numpy as jnp
from jax import lax
from jax.experimental import pallas as pl
from jax.experimental.pallas import tpu as pltpu
from jax.experimental.pallas import tpu_sc as plsc

N = 10000
E = 320000
D_FEAT = 128
H1 = 32
H2 = 16

NC = 2
NS = 16
NW = NC * NS

CH = 128
EPW = 10240
K = EPW // CH
EPAD = NW * EPW
NPAD = 10240
RB = NPAD // NS

_mesh = plsc.VectorSubcoreMesh(
    core_axis_name="c", subcore_axis_name="s", num_cores=NC, num_subcores=NS
)
_sc_params = pltpu.CompilerParams(use_tc_tiling_on_sc=False)


@functools.partial(
    pl.kernel,
    out_type=jax.ShapeDtypeStruct((NC, NPAD, 16), jnp.float32),
    mesh=_mesh,
    scratch_types=[
        pltpu.VMEM((K, CH), jnp.int32),
        pltpu.VMEM((CH, 16), jnp.float32),
        pltpu.VMEM((RB, 16), jnp.float32),
        pltpu.VMEM_SHARED((NPAD, 16), jnp.float32),
    ],
    compiler_params=_sc_params,
)
def _deg_kernel(dsts, ones_hbm, zeros_hbm, out, dst_v, ones_v, buf, acc):
    c = lax.axis_index("c")
    s = lax.axis_index("s")
    w = c * NS + s
    r0 = s * RB
    pltpu.sync_copy(zeros_hbm.at[pl.ds(r0, RB)], buf)
    pltpu.sync_copy(buf, acc.at[pl.ds(r0, RB)])
    pltpu.sync_copy(ones_hbm, ones_v)
    pltpu.sync_copy(dsts.at[w], dst_v)
    plsc.subcore_barrier()

    def step(j, carry):
        pltpu.sync_copy(ones_v, acc.at[dst_v.at[j]], add=True)
        return carry

    lax.fori_loop(0, K, step, 0)
    plsc.subcore_barrier()
    pltpu.sync_copy(acc.at[pl.ds(r0, RB)], buf)
    pltpu.sync_copy(buf, out.at[c, pl.ds(r0, RB)])


@functools.partial(
    pl.kernel,
    out_type=jax.ShapeDtypeStruct((NC, NPAD, H1), jnp.float32),
    mesh=_mesh,
    scratch_types=[
        pltpu.VMEM((K, CH), jnp.int32),
        pltpu.VMEM((K, CH), jnp.int32),
        pltpu.VMEM((2, CH, H1), jnp.float32),
        pltpu.VMEM((RB, H1), jnp.float32),
        pltpu.VMEM_SHARED((NPAD, H1), jnp.float32),
        pltpu.SemaphoreType.DMA,
        pltpu.SemaphoreType.DMA,
    ],
    compiler_params=_sc_params,
)
def _prop_kernel(hp, srcs, dsts, zeros_hbm, out, src_v, dst_v, ring, buf, acc, sem0, sem1):
    c = lax.axis_index("c")
    s = lax.axis_index("s")
    w = c * NS + s
    r0 = s * RB
    pltpu.sync_copy(zeros_hbm.at[pl.ds(r0, RB)], buf)
    pltpu.sync_copy(buf, acc.at[pl.ds(r0, RB)])
    pltpu.sync_copy(srcs.at[w], src_v)
    pltpu.sync_copy(dsts.at[w], dst_v)
    plsc.subcore_barrier()

    rows0 = ring.at[0]
    rows1 = ring.at[1]
    pltpu.async_copy(hp.at[src_v.at[0]], rows0, sem0)
    pltpu.async_copy(hp.at[src_v.at[1]], rows1, sem1)

    def step(t, carry):
        j = 2 * t
        pltpu.make_async_copy(hp.at[src_v.at[j]], rows0, sem0).wait()
        pltpu.sync_copy(rows0, acc.at[dst_v.at[j]], add=True)

        @pl.when(j + 2 < K)
        def _():
            pltpu.async_copy(hp.at[src_v.at[j + 2]], rows0, sem0)

        pltpu.make_async_copy(hp.at[src_v.at[j + 1]], rows1, sem1).wait()
        pltpu.sync_copy(rows1, acc.at[dst_v.at[j + 1]], add=True)

        @pl.when(j + 3 < K)
        def _():
            pltpu.async_copy(hp.at[src_v.at[j + 3]], rows1, sem1)

        return carry

    lax.fori_loop(0, K // 2, step, 0)
    plsc.subcore_barrier()
    pltpu.sync_copy(acc.at[pl.ds(r0, RB)], buf)
    pltpu.sync_copy(buf, out.at[c, pl.ds(r0, RB)])


_RBLK = 2000

_spec_p0 = pl.BlockSpec((1, _RBLK, 16), lambda i: (0, i, 0))
_spec_p1 = pl.BlockSpec((1, _RBLK, 16), lambda i: (1, i, 0))
_spec_s0 = pl.BlockSpec((1, _RBLK, H1), lambda i: (0, i, 0))
_spec_s1 = pl.BlockSpec((1, _RBLK, H1), lambda i: (1, i, 0))


def _dinv_of(p0_ref, p1_ref):
    return lax.rsqrt(p0_ref[0][:, :1] + p1_ref[0][:, :1] + 1.0)


def _tc_scale_body(x_ref, w1_ref, p0_ref, p1_ref, yp_ref):
    dinv = _dinv_of(p0_ref, p1_ref)
    y = jnp.dot(x_ref[...], w1_ref[...], preferred_element_type=jnp.float32)
    yp_ref[...] = y * dinv


def _tc_scale(x, w1, degp):
    grid = (N // _RBLK,)
    return pl.pallas_call(
        _tc_scale_body,
        grid=grid,
        in_specs=[
            pl.BlockSpec((_RBLK, D_FEAT), lambda i: (i, 0)),
            pl.BlockSpec((D_FEAT, H1), lambda i: (0, 0)),
            _spec_p0,
            _spec_p1,
        ],
        out_specs=pl.BlockSpec((_RBLK, H1), lambda i: (i, 0)),
        out_shape=jax.ShapeDtypeStruct((N, H1), jnp.float32),
    )(x, w1, degp, degp)


def _tc_hidden_body(s0_ref, s1_ref, yp_ref, p0_ref, p1_ref, h_ref):
    dinv = _dinv_of(p0_ref, p1_ref)
    pre = dinv * (s0_ref[0] + s1_ref[0] + yp_ref[...])
    h_ref[...] = dinv * jnp.maximum(pre, 0.0)


def _tc_hidden(s1, yp, degp):
    grid = (N // _RBLK,)
    spec32 = pl.BlockSpec((_RBLK, H1), lambda i: (i, 0))
    return pl.pallas_call(
        _tc_hidden_body,
        grid=grid,
        in_specs=[_spec_s0, _spec_s1, spec32, _spec_p0, _spec_p1],
        out_specs=spec32,
        out_shape=jax.ShapeDtypeStruct((N, H1), jnp.float32),
    )(s1, s1, yp, degp, degp)


def _tc_z_body(s0_ref, s1_ref, hp_ref, p0_ref, p1_ref, w2_ref, w3_ref, eps_ref, z_ref):
    dinv = _dinv_of(p0_ref, p1_ref)
    g = dinv * (s0_ref[0] + s1_ref[0] + hp_ref[...])
    zm = jnp.dot(g, w2_ref[...], preferred_element_type=jnp.float32)
    zl = jnp.dot(g, w3_ref[...], preferred_element_type=jnp.float32)
    z_ref[...] = zm + eps_ref[...] * jnp.exp(zl)


def _tc_z(s2, hp, degp, w2, w3, eps):
    grid = (N // _RBLK,)
    spec32 = pl.BlockSpec((_RBLK, H1), lambda i: (i, 0))
    spec16 = pl.BlockSpec((_RBLK, H2), lambda i: (i, 0))
    specw = pl.BlockSpec((H1, H2), lambda i: (0, 0))
    return pl.pallas_call(
        _tc_z_body,
        grid=grid,
        in_specs=[_spec_s0, _spec_s1, spec32, _spec_p0, _spec_p1, specw, specw, spec16],
        out_specs=spec16,
        out_shape=jax.ShapeDtypeStruct((N, H2), jnp.float32),
    )(s2, s2, hp, degp, degp, w2, w3, eps)


_DR = 400


def _dec_body(zr_ref, zc_ref, o_ref):
    o_ref[...] = lax.dot_general(
        zr_ref[...], zc_ref[...],
        dimension_numbers=(((1,), (1,)), ((), ())),
        preferred_element_type=jnp.float32,
    )


def _decoder(z):
    grid = (N // _DR,)
    return pl.pallas_call(
        _dec_body,
        grid=grid,
        in_specs=[
            pl.BlockSpec((_DR, H2), lambda i: (i, 0)),
            pl.BlockSpec((N, H2), lambda i: (0, 0)),
        ],
        out_specs=pl.BlockSpec((_DR, N), lambda i: (i, 0)),
        out_shape=jax.ShapeDtypeStruct((N, N), jnp.float32),
    )(z, z)


def kernel(features, edge_index, eps, W1, W2, W3):
    src = edge_index[0]
    dst = edge_index[1]
    pad_src = jnp.zeros((EPAD - E,), dtype=jnp.int32)
    pad_dst = N + jnp.arange(EPAD - E, dtype=jnp.int32) % (NPAD - N)
    srcs = jnp.concatenate([src, pad_src]).reshape(NW, K, CH)
    dsts = jnp.concatenate([dst, pad_dst]).reshape(NW, K, CH)

    ones16 = jnp.ones((CH, 16), jnp.float32)
    zeros16 = jnp.zeros((NPAD, 16), jnp.float32)
    zeros32 = jnp.zeros((NPAD, H1), jnp.float32)

    degp = _deg_kernel(dsts, ones16, zeros16)
    yp = _tc_scale(features, W1, degp)
    s1 = _prop_kernel(yp, srcs, dsts, zeros32)
    hp = _tc_hidden(s1, yp, degp)
    s2 = _prop_kernel(hp, srcs, dsts, zeros32)
    z = _tc_z(s2, hp, degp, W2, W3, eps)
    return _decoder(z).reshape(-1)


# speedup vs baseline: 1.2390x; 1.2390x over previous
import functools

import jax
import jax.numpy as jnp
from jax import lax
from jax.experimental import pallas as pl
from jax.experimental.pallas import tpu as pltpu
from jax.experimental.pallas import tpu_sc as plsc

N = 10000
E = 320000
D_FEAT = 128
H1 = 32
H2 = 16

NC = 2
NS = 16
NW = NC * NS

CH = 128
EPW = 10240
K = EPW // CH
EPAD = NW * EPW
NPAD = 10240
RB = NPAD // NS

_mesh = plsc.VectorSubcoreMesh(
    core_axis_name="c", subcore_axis_name="s", num_cores=NC, num_subcores=NS
)
_sc_params = pltpu.CompilerParams(use_tc_tiling_on_sc=False)


@functools.partial(
    pl.kernel,
    out_type=jax.ShapeDtypeStruct((NC, NPAD, 16), jnp.float32),
    mesh=_mesh,
    scratch_types=[
        pltpu.VMEM((K, CH), jnp.int32),
        pltpu.VMEM((CH, 16), jnp.float32),
        pltpu.VMEM((RB, 16), jnp.float32),
        pltpu.VMEM_SHARED((NPAD, 16), jnp.float32),
    ],
    compiler_params=_sc_params,
)
def _deg_kernel(dsts, ones_hbm, zeros_hbm, out, dst_v, ones_v, buf, acc):
    c = lax.axis_index("c")
    s = lax.axis_index("s")
    w = c * NS + s
    r0 = s * RB
    pltpu.sync_copy(zeros_hbm.at[pl.ds(r0, RB)], buf)
    pltpu.sync_copy(buf, acc.at[pl.ds(r0, RB)])
    pltpu.sync_copy(ones_hbm, ones_v)
    pltpu.sync_copy(dsts.at[w], dst_v)
    plsc.subcore_barrier()

    def step(j, carry):
        pltpu.sync_copy(ones_v, acc.at[dst_v.at[j]], add=True)
        return carry

    lax.fori_loop(0, K, step, 0)
    plsc.subcore_barrier()
    pltpu.sync_copy(acc.at[pl.ds(r0, RB)], buf)
    pltpu.sync_copy(buf, out.at[c, pl.ds(r0, RB)])


@functools.partial(
    pl.kernel,
    out_type=jax.ShapeDtypeStruct((NC, NPAD, H1), jnp.float32),
    mesh=_mesh,
    scratch_types=[
        pltpu.VMEM((K, CH), jnp.int32),
        pltpu.VMEM((K, CH), jnp.int32),
        pltpu.VMEM((2, CH, H1), jnp.float32),
        pltpu.VMEM((RB, H1), jnp.float32),
        pltpu.VMEM_SHARED((NPAD, H1), jnp.float32),
        pltpu.SemaphoreType.DMA,
        pltpu.SemaphoreType.DMA,
    ],
    compiler_params=_sc_params,
)
def _prop_kernel(hp, srcs, dsts, zeros_hbm, out, src_v, dst_v, ring, buf, acc, sem0, sem1):
    c = lax.axis_index("c")
    s = lax.axis_index("s")
    w = c * NS + s
    r0 = s * RB
    pltpu.sync_copy(zeros_hbm.at[pl.ds(r0, RB)], buf)
    pltpu.sync_copy(buf, acc.at[pl.ds(r0, RB)])
    pltpu.sync_copy(srcs.at[w], src_v)
    pltpu.sync_copy(dsts.at[w], dst_v)
    plsc.subcore_barrier()

    rows0 = ring.at[0]
    rows1 = ring.at[1]
    pltpu.async_copy(hp.at[src_v.at[0]], rows0, sem0)
    pltpu.async_copy(hp.at[src_v.at[1]], rows1, sem1)

    def step(t, carry):
        j = 2 * t
        pltpu.make_async_copy(hp.at[src_v.at[j]], rows0, sem0).wait()
        pltpu.sync_copy(rows0, acc.at[dst_v.at[j]], add=True)

        @pl.when(j + 2 < K)
        def _():
            pltpu.async_copy(hp.at[src_v.at[j + 2]], rows0, sem0)

        pltpu.make_async_copy(hp.at[src_v.at[j + 1]], rows1, sem1).wait()
        pltpu.sync_copy(rows1, acc.at[dst_v.at[j + 1]], add=True)

        @pl.when(j + 3 < K)
        def _():
            pltpu.async_copy(hp.at[src_v.at[j + 3]], rows1, sem1)

        return carry

    lax.fori_loop(0, K // 2, step, 0)
    plsc.subcore_barrier()
    pltpu.sync_copy(acc.at[pl.ds(r0, RB)], buf)
    pltpu.sync_copy(buf, out.at[c, pl.ds(r0, RB)])


_RBLK = 2000

_spec_p0 = pl.BlockSpec((1, _RBLK, 16), lambda i: (0, i, 0))
_spec_p1 = pl.BlockSpec((1, _RBLK, 16), lambda i: (1, i, 0))
_spec_s0 = pl.BlockSpec((1, _RBLK, H1), lambda i: (0, i, 0))
_spec_s1 = pl.BlockSpec((1, _RBLK, H1), lambda i: (1, i, 0))


def _dinv_of(p0_ref, p1_ref):
    return lax.rsqrt(p0_ref[0][:, :1] + p1_ref[0][:, :1] + 1.0)


def _tc_scale_body(x_ref, w1_ref, p0_ref, p1_ref, yp_ref):
    dinv = _dinv_of(p0_ref, p1_ref)
    y = jnp.dot(x_ref[...], w1_ref[...], preferred_element_type=jnp.float32)
    yp_ref[...] = y * dinv


def _tc_scale(x, w1, degp):
    grid = (N // _RBLK,)
    return pl.pallas_call(
        _tc_scale_body,
        grid=grid,
        in_specs=[
            pl.BlockSpec((_RBLK, D_FEAT), lambda i: (i, 0)),
            pl.BlockSpec((D_FEAT, H1), lambda i: (0, 0)),
            _spec_p0,
            _spec_p1,
        ],
        out_specs=pl.BlockSpec((_RBLK, H1), lambda i: (i, 0)),
        out_shape=jax.ShapeDtypeStruct((N, H1), jnp.float32),
    )(x, w1, degp, degp)


def _tc_hidden_body(s0_ref, s1_ref, yp_ref, p0_ref, p1_ref, h_ref):
    dinv = _dinv_of(p0_ref, p1_ref)
    pre = dinv * (s0_ref[0] + s1_ref[0] + yp_ref[...])
    h_ref[...] = dinv * jnp.maximum(pre, 0.0)


def _tc_hidden(s1, yp, degp):
    grid = (N // _RBLK,)
    spec32 = pl.BlockSpec((_RBLK, H1), lambda i: (i, 0))
    return pl.pallas_call(
        _tc_hidden_body,
        grid=grid,
        in_specs=[_spec_s0, _spec_s1, spec32, _spec_p0, _spec_p1],
        out_specs=spec32,
        out_shape=jax.ShapeDtypeStruct((N, H1), jnp.float32),
    )(s1, s1, yp, degp, degp)


def _tc_z_body(s0_ref, s1_ref, hp_ref, p0_ref, p1_ref, w2_ref, w3_ref, eps_ref, z_ref):
    dinv = _dinv_of(p0_ref, p1_ref)
    g = dinv * (s0_ref[0] + s1_ref[0] + hp_ref[...])
    zm = jnp.dot(g, w2_ref[...], preferred_element_type=jnp.float32)
    zl = jnp.dot(g, w3_ref[...], preferred_element_type=jnp.float32)
    z_ref[...] = zm + eps_ref[...] * jnp.exp(zl)


def _tc_z(s2, hp, degp, w2, w3, eps):
    grid = (N // _RBLK,)
    spec32 = pl.BlockSpec((_RBLK, H1), lambda i: (i, 0))
    spec16 = pl.BlockSpec((_RBLK, H2), lambda i: (i, 0))
    specw = pl.BlockSpec((H1, H2), lambda i: (0, 0))
    return pl.pallas_call(
        _tc_z_body,
        grid=grid,
        in_specs=[_spec_s0, _spec_s1, spec32, _spec_p0, _spec_p1, specw, specw, spec16],
        out_specs=spec16,
        out_shape=jax.ShapeDtypeStruct((N, H2), jnp.float32),
    )(s2, s2, hp, degp, degp, w2, w3, eps)


_DR = 400


def _dec_body(zr_ref, zc_ref, o_ref):
    o_ref[...] = lax.dot_general(
        zr_ref[...], zc_ref[...],
        dimension_numbers=(((1,), (1,)), ((), ())),
        preferred_element_type=jnp.float32,
    )


def _decoder(z):
    grid = (N // _DR,)
    return pl.pallas_call(
        _dec_body,
        grid=grid,
        in_specs=[
            pl.BlockSpec((_DR, H2), lambda i: (i, 0)),
            pl.BlockSpec((N, H2), lambda i: (0, 0)),
        ],
        out_specs=pl.BlockSpec((_DR, N), lambda i: (i, 0)),
        out_shape=jax.ShapeDtypeStruct((N, N), jnp.float32),
    )(z, z)


def kernel(features, edge_index, eps, W1, W2, W3):
    src = edge_index[0]
    dst = edge_index[1]
    pad_src = jnp.arange(EPAD - E, dtype=jnp.int32) % N
    pad_dst = N + jnp.arange(EPAD - E, dtype=jnp.int32) % (NPAD - N)
    srcs = jnp.concatenate([src, pad_src]).reshape(NW, K, CH)
    dsts = jnp.concatenate([dst, pad_dst]).reshape(NW, K, CH)

    ones16 = jnp.ones((CH, 16), jnp.float32)
    zeros16 = jnp.zeros((NPAD, 16), jnp.float32)
    zeros32 = jnp.zeros((NPAD, H1), jnp.float32)

    degp = _deg_kernel(dsts, ones16, zeros16)
    yp = _tc_scale(features, W1, degp)
    s1 = _prop_kernel(yp, srcs, dsts, zeros32)
    hp = _tc_hidden(s1, yp, degp)
    s2 = _prop_kernel(hp, srcs, dsts, zeros32)
    z = _tc_z(s2, hp, degp, W2, W3, eps)
    return _decoder(z).reshape(-1)


# speedup vs baseline: 1.2864x; 1.0383x over previous
import functools

import jax
import jax.numpy as jnp
from jax import lax
from jax.experimental import pallas as pl
from jax.experimental.pallas import tpu as pltpu
from jax.experimental.pallas import tpu_sc as plsc

N = 10000
E = 320000
D_FEAT = 128
H1 = 32
H2 = 16

NC = 2
NS = 16
NW = NC * NS

CH = 128
EPW = 10240
K = EPW // CH
EPAD = NW * EPW
NPAD = 10240
RB = NPAD // NS

_mesh = plsc.VectorSubcoreMesh(
    core_axis_name="c", subcore_axis_name="s", num_cores=NC, num_subcores=NS
)
_sc_params = pltpu.CompilerParams(use_tc_tiling_on_sc=False)


@functools.partial(
    pl.kernel,
    out_type=jax.ShapeDtypeStruct((NC, NPAD, 16), jnp.float32),
    mesh=_mesh,
    scratch_types=[
        pltpu.VMEM((K, CH), jnp.int32),
        pltpu.VMEM((CH, 16), jnp.float32),
        pltpu.VMEM((RB, 16), jnp.float32),
        pltpu.VMEM_SHARED((NPAD, 16), jnp.float32),
    ],
    compiler_params=_sc_params,
)
def _deg_kernel(dsts, ones_hbm, zeros_hbm, out, dst_v, ones_v, buf, acc):
    c = lax.axis_index("c")
    s = lax.axis_index("s")
    w = c * NS + s
    r0 = s * RB
    pltpu.sync_copy(zeros_hbm.at[pl.ds(r0, RB)], buf)
    pltpu.sync_copy(buf, acc.at[pl.ds(r0, RB)])
    pltpu.sync_copy(ones_hbm, ones_v)
    pltpu.sync_copy(dsts.at[w], dst_v)
    plsc.subcore_barrier()

    def step(j, carry):
        pltpu.sync_copy(ones_v, acc.at[dst_v.at[j]], add=True)
        return carry

    lax.fori_loop(0, K, step, 0)
    plsc.subcore_barrier()
    pltpu.sync_copy(acc.at[pl.ds(r0, RB)], buf)
    pltpu.sync_copy(buf, out.at[c, pl.ds(r0, RB)])


@functools.partial(
    pl.kernel,
    out_type=jax.ShapeDtypeStruct((NC, NPAD, H1), jnp.float32),
    mesh=_mesh,
    scratch_types=[
        pltpu.VMEM((K, CH), jnp.int32),
        pltpu.VMEM((K, CH), jnp.int32),
        pltpu.VMEM((4, CH, H1), jnp.float32),
        pltpu.VMEM((RB, H1), jnp.float32),
        pltpu.VMEM_SHARED((NPAD, H1), jnp.float32),
        [pltpu.SemaphoreType.DMA] * 4,
        [pltpu.SemaphoreType.DMA] * 4,
    ],
    compiler_params=_sc_params,
)
def _prop_kernel(hp, srcs, dsts, zeros_hbm, out, src_v, dst_v, ring, buf, acc, gsem, ssem):
    c = lax.axis_index("c")
    s = lax.axis_index("s")
    w = c * NS + s
    r0 = s * RB
    pltpu.sync_copy(zeros_hbm.at[pl.ds(r0, RB)], buf)
    pltpu.sync_copy(buf, acc.at[pl.ds(r0, RB)])
    pltpu.sync_copy(srcs.at[w], src_v)
    pltpu.sync_copy(dsts.at[w], dst_v)
    plsc.subcore_barrier()

    NB = 4
    bufs = [ring.at[b] for b in range(NB)]
    for b in range(NB):
        pltpu.async_copy(hp.at[src_v.at[b]], bufs[b], gsem[b])

    def step(t, carry):
        j0 = NB * t
        for b in range(NB):
            pltpu.make_async_copy(hp.at[src_v.at[j0 + b]], bufs[b], gsem[b]).wait()
            pltpu.async_copy(bufs[b], acc.at[dst_v.at[j0 + b]], ssem[b], add=True)
        for b in range(NB):
            @pl.when(j0 + b + NB < K)
            def _(b=b):
                pltpu.make_async_copy(bufs[b], acc.at[dst_v.at[j0 + b]], ssem[b]).wait()
                pltpu.async_copy(hp.at[src_v.at[j0 + b + NB]], bufs[b], gsem[b])
        return carry

    lax.fori_loop(0, K // NB, step, 0)
    for b in range(NB):
        pltpu.make_async_copy(bufs[b], acc.at[dst_v.at[K - NB + b]], ssem[b]).wait()
    plsc.subcore_barrier()
    pltpu.sync_copy(acc.at[pl.ds(r0, RB)], buf)
    pltpu.sync_copy(buf, out.at[c, pl.ds(r0, RB)])


_RBLK = 2000

_spec_p0 = pl.BlockSpec((1, _RBLK, 16), lambda i: (0, i, 0))
_spec_p1 = pl.BlockSpec((1, _RBLK, 16), lambda i: (1, i, 0))
_spec_s0 = pl.BlockSpec((1, _RBLK, H1), lambda i: (0, i, 0))
_spec_s1 = pl.BlockSpec((1, _RBLK, H1), lambda i: (1, i, 0))


def _dinv_of(p0_ref, p1_ref):
    return lax.rsqrt(p0_ref[0][:, :1] + p1_ref[0][:, :1] + 1.0)


def _tc_scale_body(x_ref, w1_ref, p0_ref, p1_ref, yp_ref):
    dinv = _dinv_of(p0_ref, p1_ref)
    y = jnp.dot(x_ref[...], w1_ref[...], preferred_element_type=jnp.float32)
    yp_ref[...] = y * dinv


def _tc_scale(x, w1, degp):
    grid = (N // _RBLK,)
    return pl.pallas_call(
        _tc_scale_body,
        grid=grid,
        in_specs=[
            pl.BlockSpec((_RBLK, D_FEAT), lambda i: (i, 0)),
            pl.BlockSpec((D_FEAT, H1), lambda i: (0, 0)),
            _spec_p0,
            _spec_p1,
        ],
        out_specs=pl.BlockSpec((_RBLK, H1), lambda i: (i, 0)),
        out_shape=jax.ShapeDtypeStruct((N, H1), jnp.float32),
    )(x, w1, degp, degp)


def _tc_hidden_body(s0_ref, s1_ref, yp_ref, p0_ref, p1_ref, h_ref):
    dinv = _dinv_of(p0_ref, p1_ref)
    pre = dinv * (s0_ref[0] + s1_ref[0] + yp_ref[...])
    h_ref[...] = dinv * jnp.maximum(pre, 0.0)


def _tc_hidden(s1, yp, degp):
    grid = (N // _RBLK,)
    spec32 = pl.BlockSpec((_RBLK, H1), lambda i: (i, 0))
    return pl.pallas_call(
        _tc_hidden_body,
        grid=grid,
        in_specs=[_spec_s0, _spec_s1, spec32, _spec_p0, _spec_p1],
        out_specs=spec32,
        out_shape=jax.ShapeDtypeStruct((N, H1), jnp.float32),
    )(s1, s1, yp, degp, degp)


def _tc_z_body(s0_ref, s1_ref, hp_ref, p0_ref, p1_ref, w2_ref, w3_ref, eps_ref, z_ref):
    dinv = _dinv_of(p0_ref, p1_ref)
    g = dinv * (s0_ref[0] + s1_ref[0] + hp_ref[...])
    zm = jnp.dot(g, w2_ref[...], preferred_element_type=jnp.float32)
    zl = jnp.dot(g, w3_ref[...], preferred_element_type=jnp.float32)
    z_ref[...] = zm + eps_ref[...] * jnp.exp(zl)


def _tc_z(s2, hp, degp, w2, w3, eps):
    grid = (N // _RBLK,)
    spec32 = pl.BlockSpec((_RBLK, H1), lambda i: (i, 0))
    spec16 = pl.BlockSpec((_RBLK, H2), lambda i: (i, 0))
    specw = pl.BlockSpec((H1, H2), lambda i: (0, 0))
    return pl.pallas_call(
        _tc_z_body,
        grid=grid,
        in_specs=[_spec_s0, _spec_s1, spec32, _spec_p0, _spec_p1, specw, specw, spec16],
        out_specs=spec16,
        out_shape=jax.ShapeDtypeStruct((N, H2), jnp.float32),
    )(s2, s2, hp, degp, degp, w2, w3, eps)


_DR = 400


def _dec_body(zr_ref, zc_ref, o_ref):
    o_ref[...] = lax.dot_general(
        zr_ref[...], zc_ref[...],
        dimension_numbers=(((1,), (1,)), ((), ())),
        preferred_element_type=jnp.float32,
    )


def _decoder(z):
    grid = (N // _DR,)
    return pl.pallas_call(
        _dec_body,
        grid=grid,
        in_specs=[
            pl.BlockSpec((_DR, H2), lambda i: (i, 0)),
            pl.BlockSpec((N, H2), lambda i: (0, 0)),
        ],
        out_specs=pl.BlockSpec((_DR, N), lambda i: (i, 0)),
        out_shape=jax.ShapeDtypeStruct((N, N), jnp.float32),
    )(z, z)


def kernel(features, edge_index, eps, W1, W2, W3):
    src = edge_index[0]
    dst = edge_index[1]
    pad_src = jnp.arange(EPAD - E, dtype=jnp.int32) % N
    pad_dst = N + jnp.arange(EPAD - E, dtype=jnp.int32) % (NPAD - N)
    srcs = jnp.concatenate([src, pad_src]).reshape(NW, K, CH)
    dsts = jnp.concatenate([dst, pad_dst]).reshape(NW, K, CH)

    ones16 = jnp.ones((CH, 16), jnp.float32)
    zeros16 = jnp.zeros((NPAD, 16), jnp.float32)
    zeros32 = jnp.zeros((NPAD, H1), jnp.float32)

    degp = _deg_kernel(dsts, ones16, zeros16)
    yp = _tc_scale(features, W1, degp)
    s1 = _prop_kernel(yp, srcs, dsts, zeros32)
    hp = _tc_hidden(s1, yp, degp)
    s2 = _prop_kernel(hp, srcs, dsts, zeros32)
    z = _tc_z(s2, hp, degp, W2, W3, eps)
    return _decoder(z).reshape(-1)
